# baseline (speedup 1.0000x reference)
import functools

import jax
import jax.numpy as jnp
from jax import lax
from jax.experimental import pallas as pl
from jax.experimental.pallas import tpu as pltpu
from jax.experimental.pallas import tpu_sc as plsc

B = 16384
D = 64
NUM_CORES = 2
NUM_SUBCORES = 16
NW = NUM_CORES * NUM_SUBCORES
BPW = B // NW
MARGIN = 0.1


def _cf_body(uid_hbm, mid_hbm, ue_hbm, me_hbm, bu_hbm, bm_hbm, out_hbm,
             uidx_v, midx_v, u_rows, m_rows, bu_v, bm_v, out_v, sem):
    wid = lax.axis_index("s") * NUM_CORES + lax.axis_index("c")
    base = wid * BPW

    pltpu.sync_copy(uid_hbm.at[pl.ds(base, BPW)], uidx_v)
    pltpu.sync_copy(mid_hbm.at[pl.ds(base, BPW)], midx_v)

    cu = pltpu.async_copy(ue_hbm.at[uidx_v], u_rows, sem)
    cm = pltpu.async_copy(me_hbm.at[midx_v], m_rows, sem)
    cbu = pltpu.async_copy(bu_hbm.at[uidx_v], bu_v, sem)
    cbm = pltpu.async_copy(bm_hbm.at[midx_v], bm_v, sem)
    cu.wait()
    cm.wait()
    cbu.wait()
    cbm.wait()

    lane = lax.iota(jnp.int32, 16)

    def chunk_body(c, carry):
        o = c * 16
        acc = jnp.zeros((16,), jnp.float32)
        for k in range(16):
            r = o + k
            p = (u_rows[r, pl.ds(0, 16)] * m_rows[r, pl.ds(0, 16)]
                 + u_rows[r, pl.ds(16, 16)] * m_rows[r, pl.ds(16, 16)]
                 + u_rows[r, pl.ds(32, 16)] * m_rows[r, pl.ds(32, 16)]
                 + u_rows[r, pl.ds(48, 16)] * m_rows[r, pl.ds(48, 16)])
            acc = jnp.where(lane == k, jnp.sum(p), acc)
        x = acc + bu_v[pl.ds(o, 16)] + bm_v[pl.ds(o, 16)]
        y = 1.0 / (1.0 + jnp.exp(-x))
        out_v[pl.ds(o, 16)] = y * (1.0 + 2.0 * MARGIN) - MARGIN
        return carry

    lax.fori_loop(0, BPW // 16, chunk_body, 0)

    pltpu.sync_copy(out_v, out_hbm.at[pl.ds(base, BPW)])


@functools.partial(
    pl.kernel,
    out_type=jax.ShapeDtypeStruct((B,), jnp.float32),
    mesh=plsc.VectorSubcoreMesh(core_axis_name="c", subcore_axis_name="s"),
    compiler_params=pltpu.CompilerParams(
        needs_layout_passes=False, use_tc_tiling_on_sc=False),
    scratch_types=[
        pltpu.VMEM((BPW,), jnp.int32),
        pltpu.VMEM((BPW,), jnp.int32),
        pltpu.VMEM((BPW, D), jnp.float32),
        pltpu.VMEM((BPW, D), jnp.float32),
        pltpu.VMEM((BPW,), jnp.float32),
        pltpu.VMEM((BPW,), jnp.float32),
        pltpu.VMEM((BPW,), jnp.float32),
        pltpu.SemaphoreType.DMA,
    ],
)
def _cf_kernel(*args):
    _cf_body(*args)


def kernel(user_id, movie_id, emb_users, emb_movies, bias_user, bias_movie):
    return _cf_kernel(
        user_id.astype(jnp.int32),
        movie_id.astype(jnp.int32),
        emb_users,
        emb_movies,
        bias_user.reshape(-1),
        bias_movie.reshape(-1),
    )
